# baseline (device time: 11688 ns/iter reference)
import jax
import jax.numpy as jnp
from jax import lax
from jax.experimental import pallas as pl
from jax.experimental.pallas import tpu as pltpu

N_DEV = 8
M = 512
N = 512
M_PER = M // N_DEV


def kernel(A, B):
    def body(a_hbm, b_hbm, out_ref, a_ref, b_ref, p_ref, comm_ref,
             send_sems, recv_sems, ready_sems, in_sems):
        me = lax.axis_index("i")

        barrier = pltpu.get_barrier_semaphore()
        for nbr in [(me + 1) % N_DEV, (me - 1) % N_DEV]:
            pl.semaphore_signal(
                barrier, inc=1,
                device_id=(nbr,), device_id_type=pl.DeviceIdType.MESH,
            )

        for d in range(1, N_DEV):
            pl.semaphore_signal(
                ready_sems.at[d], inc=1,
                device_id=((me - d) % N_DEV,),
                device_id_type=pl.DeviceIdType.MESH,
            )

        a_copy = pltpu.make_async_copy(a_hbm, a_ref, in_sems.at[0])
        b_copy = pltpu.make_async_copy(b_hbm, b_ref, in_sems.at[1])
        a_copy.start()
        b_copy.start()
        a_copy.wait()
        b_copy.wait()

        b16 = b_ref[...].astype(jnp.bfloat16)

        rdmas = []
        for d in range(1, N_DEV):
            tgt = (me + d) % N_DEV
            rows = pl.ds(tgt * M_PER, M_PER)
            p_ref[rows, :] = jnp.dot(
                a_ref[rows, :].astype(jnp.bfloat16),
                b16,
                preferred_element_type=jnp.float32,
            ).astype(jnp.bfloat16)
            pl.semaphore_wait(ready_sems.at[d], 1)
            rdma = pltpu.make_async_remote_copy(
                src_ref=p_ref.at[rows, :],
                dst_ref=comm_ref.at[d],
                send_sem=send_sems.at[d],
                recv_sem=recv_sems.at[d],
                device_id=(tgt,),
                device_id_type=pl.DeviceIdType.MESH,
            )
            rdma.start()
            rdmas.append(rdma)

        own_rows = pl.ds(me * M_PER, M_PER)
        p_ref[own_rows, :] = jnp.dot(
            a_ref[own_rows, :].astype(jnp.bfloat16),
            b16,
            preferred_element_type=jnp.float32,
        ).astype(jnp.bfloat16)

        acc = p_ref[own_rows, :].astype(jnp.float32)
        for d in range(1, N_DEV):
            rdmas[d - 1].wait_recv()
            acc = acc + comm_ref[d].astype(jnp.float32)

        pl.semaphore_wait(barrier, 2)
        out_ref[...] = acc

        for d in range(1, N_DEV):
            rdmas[d - 1].wait_send()

    return pl.pallas_call(
        body,
        out_shape=jax.ShapeDtypeStruct((M_PER, N), jnp.float32),
        in_specs=[
            pl.BlockSpec(memory_space=pl.ANY),
            pl.BlockSpec(memory_space=pl.ANY),
        ],
        out_specs=pl.BlockSpec(memory_space=pltpu.VMEM),
        scratch_shapes=[
            pltpu.VMEM(A.shape, jnp.float32),
            pltpu.VMEM(B.shape, jnp.float32),
            pltpu.VMEM((M, N), jnp.bfloat16),
            pltpu.VMEM((N_DEV, M_PER, N), jnp.bfloat16),
            pltpu.SemaphoreType.DMA((N_DEV,)),
            pltpu.SemaphoreType.DMA((N_DEV,)),
            pltpu.SemaphoreType.REGULAR((N_DEV,)),
            pltpu.SemaphoreType.DMA((2,)),
        ],
        compiler_params=pltpu.CompilerParams(collective_id=0),
    )(A, B)


# device time: 11637 ns/iter; 1.0044x vs baseline; 1.0044x over previous
import jax
import jax.numpy as jnp
from jax import lax
from jax.experimental import pallas as pl
from jax.experimental.pallas import tpu as pltpu

N_DEV = 8
M = 512
N = 512
M_PER = M // N_DEV


def kernel(A, B):
    def body(a_hbm, b_hbm, out_ref, a_ref, b_ref, p_ref, comm_ref,
             send_sems, recv_sems, ready_sems, in_sems):
        me = lax.axis_index("i")

        barrier = pltpu.get_barrier_semaphore()
        for nbr in [(me + 1) % N_DEV, (me - 1) % N_DEV]:
            pl.semaphore_signal(
                barrier, inc=1,
                device_id=(nbr,), device_id_type=pl.DeviceIdType.MESH,
            )

        for d in range(1, N_DEV):
            pl.semaphore_signal(
                ready_sems.at[d], inc=1,
                device_id=((me - d) % N_DEV,),
                device_id_type=pl.DeviceIdType.MESH,
            )

        a_copy = pltpu.make_async_copy(a_hbm, a_ref, in_sems.at[0])
        b_copy = pltpu.make_async_copy(b_hbm, b_ref, in_sems.at[1])
        a_copy.start()
        b_copy.start()
        a_copy.wait()
        b_copy.wait()

        b16 = b_ref[...].astype(jnp.bfloat16)

        rdmas = []
        for d in range(1, N_DEV):
            tgt = (me + d) % N_DEV
            rows = pl.ds(tgt * M_PER, M_PER)
            p_ref[rows, :] = jnp.dot(
                a_ref[rows, :].astype(jnp.bfloat16),
                b16,
                preferred_element_type=jnp.float32,
            ).astype(jnp.bfloat16)
            pl.semaphore_wait(ready_sems.at[d], 1)
            rdma = pltpu.make_async_remote_copy(
                src_ref=p_ref.at[rows, :],
                dst_ref=comm_ref.at[d],
                send_sem=send_sems.at[d],
                recv_sem=recv_sems.at[d],
                device_id=(tgt,),
                device_id_type=pl.DeviceIdType.MESH,
            )
            rdma.start()
            rdmas.append(rdma)

        own_rows = pl.ds(me * M_PER, M_PER)
        p_ref[own_rows, :] = jnp.dot(
            a_ref[own_rows, :].astype(jnp.bfloat16),
            b16,
            preferred_element_type=jnp.float32,
        ).astype(jnp.bfloat16)

        acc = p_ref[own_rows, :].astype(jnp.float32)
        for d in range(1, N_DEV):
            rdmas[d - 1].wait_recv()
            acc = acc + comm_ref[d].astype(jnp.float32)

        pl.semaphore_wait(barrier, 2)
        out_ref[...] = acc

        for d in range(1, N_DEV):
            rdmas[d - 1].wait_send()

    return pl.pallas_call(
        body,
        out_shape=jax.ShapeDtypeStruct((M_PER, N), jnp.float32),
        in_specs=[
            pl.BlockSpec(memory_space=pltpu.MemorySpace.HBM),
            pl.BlockSpec(memory_space=pltpu.MemorySpace.HBM),
        ],
        out_specs=pl.BlockSpec(memory_space=pltpu.VMEM),
        scratch_shapes=[
            pltpu.VMEM(A.shape, jnp.float32),
            pltpu.VMEM(B.shape, jnp.float32),
            pltpu.VMEM((M, N), jnp.bfloat16),
            pltpu.VMEM((N_DEV, M_PER, N), jnp.bfloat16),
            pltpu.SemaphoreType.DMA((N_DEV,)),
            pltpu.SemaphoreType.DMA((N_DEV,)),
            pltpu.SemaphoreType.REGULAR((N_DEV,)),
            pltpu.SemaphoreType.DMA((2,)),
        ],
        compiler_params=pltpu.CompilerParams(collective_id=0),
    )(A, B)
